# baseline (speedup 1.0000x reference)
import jax
import jax.numpy as jnp
from jax import lax
from jax.experimental import pallas as pl
from jax.experimental.pallas import tpu as pltpu
from jax.experimental.pallas import tpu_sc as plsc

STU_NUM = 100000
PP_DIM = 128
BATCH = 16384
X_DIM = 128
OUT_DIM = PP_DIM + X_DIM

NC = 2
NS = 16
NW = NC * NS
B_PER_W = BATCH // NW

C = 128
NCHUNK = B_PER_W // C
NBUF = 3

XCHUNK = 4
XROWS = BATCH // NC // XCHUNK


def _gather_concat(x_hbm, idx_hbm, table_hbm, out_hbm,
                   idx0, idx1, idx2, idx3, rows_v,
                   isem, gsem, rsem, xsem):
    idx_bufs = [idx0, idx1, idx2, idx3]
    cid = lax.axis_index("c")
    sid = lax.axis_index("s")
    wid = sid * NC + cid
    base = wid * B_PER_W

    def x_copy(k):
        row0 = (cid * XCHUNK + k) * XROWS
        return pltpu.make_async_copy(
            x_hbm.at[pl.ds(row0, XROWS)],
            out_hbm.at[pl.ds(row0, XROWS), pl.ds(PP_DIM, X_DIM)], xsem)

    @pl.when(sid == 0)
    def _():
        for k in range(XCHUNK):
            x_copy(k).start()

    for k in range(NCHUNK):
        pltpu.async_copy(idx_hbm.at[pl.ds(base + k * C, C)], idx_bufs[k],
                         isem)
    for k in range(NCHUNK):
        pltpu.make_async_copy(idx_hbm.at[pl.ds(base + k * C, C)],
                              idx_bufs[k], isem).wait()

    def fire(k, b):
        pltpu.async_copy(table_hbm.at[idx_bufs[k]], rows_v.at[b], gsem.at[b])

    def wait_out(k, b):
        pltpu.make_async_copy(
            rows_v.at[b],
            out_hbm.at[pl.ds(base + k * C, C), pl.ds(0, PP_DIM)],
            rsem.at[b]).wait()

    for k in range(min(NBUF, NCHUNK)):
        fire(k, k % NBUF)
    for k in range(NCHUNK):
        b = k % NBUF
        pltpu.make_async_copy(table_hbm.at[idx_bufs[k]], rows_v.at[b],
                              gsem.at[b]).wait()
        pltpu.async_copy(
            rows_v.at[b],
            out_hbm.at[pl.ds(base + k * C, C), pl.ds(0, PP_DIM)], rsem.at[b])
        nk = k + NBUF
        if nk < NCHUNK:
            wait_out(k, b)
            fire(nk, b)
    for k in range(max(0, NCHUNK - NBUF), NCHUNK):
        wait_out(k, k % NBUF)

    @pl.when(sid == 0)
    def _():
        for k in range(XCHUNK):
            x_copy(k).wait()


@jax.jit
def _run(x, stu_id, ability_emb):
    mesh = plsc.VectorSubcoreMesh(core_axis_name="c", subcore_axis_name="s")
    return pl.kernel(
        _gather_concat,
        out_type=jax.ShapeDtypeStruct((BATCH, OUT_DIM), jnp.float32),
        mesh=mesh,
        scratch_types=[
            pltpu.VMEM((C,), jnp.int32),
            pltpu.VMEM((C,), jnp.int32),
            pltpu.VMEM((C,), jnp.int32),
            pltpu.VMEM((C,), jnp.int32),
            pltpu.VMEM((NBUF, C, PP_DIM), jnp.float32),
            pltpu.SemaphoreType.DMA,
            pltpu.SemaphoreType.DMA((NBUF,)),
            pltpu.SemaphoreType.DMA((NBUF,)),
            pltpu.SemaphoreType.DMA,
        ],
    )(x, stu_id, ability_emb)


def kernel(x, stu_id, ability_emb):
    return _run(x, stu_id.astype(jnp.int32), ability_emb)


# speedup vs baseline: 8.5606x; 8.5606x over previous
import jax
import jax.numpy as jnp
from jax import lax
from jax.experimental import pallas as pl
from jax.experimental.pallas import tpu as pltpu
from jax.experimental.pallas import tpu_sc as plsc

STU_NUM = 100000
PP_DIM = 128
BATCH = 16384
X_DIM = 128
OUT_DIM = PP_DIM + X_DIM

NC = 2
NS = 16
NW = NC * NS
B_PER_W = BATCH // NW

C = 128
NCHUNK = B_PER_W // C
NBUF = 3


def _gather_concat(x_hbm, idx_hbm, table_hbm, out_hbm,
                   idx_v, rows_v, x_v, isem, insem, outsem):
    wid = lax.axis_index("s") * NC + lax.axis_index("c")
    base = wid * B_PER_W
    for k in range(NCHUNK):
        pltpu.async_copy(idx_hbm.at[pl.ds(base + k * C, C)], idx_v.at[k],
                         isem)
    for k in range(NCHUNK):
        pltpu.make_async_copy(idx_hbm.at[pl.ds(base + k * C, C)],
                              idx_v.at[k], isem).wait()

    def gather(k, b):
        return pltpu.make_async_copy(table_hbm.at[idx_v.at[k]],
                                     rows_v.at[b], insem.at[b])

    def x_load(k, b):
        return pltpu.make_async_copy(x_hbm.at[pl.ds(base + k * C, C)],
                                     x_v.at[b], insem.at[b])

    def rows_write(k, b):
        return pltpu.make_async_copy(
            rows_v.at[b],
            out_hbm.at[pl.ds(base + k * C, C), pl.ds(0, PP_DIM)],
            outsem.at[b])

    def x_write(k, b):
        return pltpu.make_async_copy(
            x_v.at[b],
            out_hbm.at[pl.ds(base + k * C, C), pl.ds(PP_DIM, X_DIM)],
            outsem.at[b])

    for k in range(min(NBUF, NCHUNK)):
        gather(k, k % NBUF).start()
        x_load(k, k % NBUF).start()
    for k in range(NCHUNK):
        b = k % NBUF
        x_load(k, b).wait()
        x_write(k, b).start()
        gather(k, b).wait()
        rows_write(k, b).start()
        nk = k + NBUF
        if nk < NCHUNK:
            x_write(k, b).wait()
            rows_write(k, b).wait()
            gather(nk, b).start()
            x_load(nk, b).start()
    for k in range(max(0, NCHUNK - NBUF), NCHUNK):
        b = k % NBUF
        x_write(k, b).wait()
        rows_write(k, b).wait()


@jax.jit
def _run(x, stu_id, ability_emb):
    mesh = plsc.VectorSubcoreMesh(core_axis_name="c", subcore_axis_name="s")
    return pl.kernel(
        _gather_concat,
        out_type=jax.ShapeDtypeStruct((BATCH, OUT_DIM), jnp.float32),
        mesh=mesh,
        scratch_types=[
            pltpu.VMEM((NCHUNK, C), jnp.int32),
            pltpu.VMEM((NBUF, C, PP_DIM), jnp.float32),
            pltpu.VMEM((NBUF, C, X_DIM), jnp.float32),
            pltpu.SemaphoreType.DMA,
            pltpu.SemaphoreType.DMA((NBUF,)),
            pltpu.SemaphoreType.DMA((NBUF,)),
        ],
    )(x, stu_id, ability_emb)


def kernel(x, stu_id, ability_emb):
    return _run(x, stu_id.astype(jnp.int32), ability_emb)


# speedup vs baseline: 8.6281x; 1.0079x over previous
import jax
import jax.numpy as jnp
from jax import lax
from jax.experimental import pallas as pl
from jax.experimental.pallas import tpu as pltpu
from jax.experimental.pallas import tpu_sc as plsc

STU_NUM = 100000
PP_DIM = 128
BATCH = 16384
X_DIM = 128
OUT_DIM = PP_DIM + X_DIM

NC = 2
NS = 16
NW = NC * NS
B_PER_W = BATCH // NW

C = 128
NCHUNK = B_PER_W // C
NBUF = 3


def _gather_concat(x_hbm, idx_hbm, table_hbm, out_hbm,
                   idx_v, rows_v, x_v, isem, gsem, xsem, rsem, wsem):
    wid = lax.axis_index("s") * NC + lax.axis_index("c")
    base = wid * B_PER_W
    for k in range(NCHUNK):
        pltpu.async_copy(idx_hbm.at[pl.ds(base + k * C, C)], idx_v.at[k],
                         isem)
    for k in range(NCHUNK):
        pltpu.make_async_copy(idx_hbm.at[pl.ds(base + k * C, C)],
                              idx_v.at[k], isem).wait()

    def gather(k, b):
        return pltpu.make_async_copy(table_hbm.at[idx_v.at[k]],
                                     rows_v.at[b], gsem.at[b])

    def x_load(k, b):
        return pltpu.make_async_copy(x_hbm.at[pl.ds(base + k * C, C)],
                                     x_v.at[b], xsem.at[b])

    def rows_write(k, b):
        return pltpu.make_async_copy(
            rows_v.at[b],
            out_hbm.at[pl.ds(base + k * C, C), pl.ds(0, PP_DIM)],
            rsem.at[b])

    def x_write(k, b):
        return pltpu.make_async_copy(
            x_v.at[b],
            out_hbm.at[pl.ds(base + k * C, C), pl.ds(PP_DIM, X_DIM)],
            wsem.at[b])

    for k in range(min(NBUF, NCHUNK)):
        gather(k, k % NBUF).start()
        x_load(k, k % NBUF).start()
    for k in range(NCHUNK):
        b = k % NBUF
        x_load(k, b).wait()
        x_write(k, b).start()
        gather(k, b).wait()
        rows_write(k, b).start()
        nk = k + NBUF
        if nk < NCHUNK:
            x_write(k, b).wait()
            rows_write(k, b).wait()
            gather(nk, b).start()
            x_load(nk, b).start()
    for k in range(max(0, NCHUNK - NBUF), NCHUNK):
        b = k % NBUF
        x_write(k, b).wait()
        rows_write(k, b).wait()


@jax.jit
def _run(x, stu_id, ability_emb):
    mesh = plsc.VectorSubcoreMesh(core_axis_name="c", subcore_axis_name="s")
    return pl.kernel(
        _gather_concat,
        out_type=jax.ShapeDtypeStruct((BATCH, OUT_DIM), jnp.float32),
        mesh=mesh,
        scratch_types=[
            pltpu.VMEM((NCHUNK, C), jnp.int32),
            pltpu.VMEM((NBUF, C, PP_DIM), jnp.float32),
            pltpu.VMEM((NBUF, C, X_DIM), jnp.float32),
            pltpu.SemaphoreType.DMA,
            pltpu.SemaphoreType.DMA((NBUF,)),
            pltpu.SemaphoreType.DMA((NBUF,)),
            pltpu.SemaphoreType.DMA((NBUF,)),
            pltpu.SemaphoreType.DMA((NBUF,)),
        ],
    )(x, stu_id, ability_emb)


def kernel(x, stu_id, ability_emb):
    return _run(x, stu_id.astype(jnp.int32), ability_emb)


# speedup vs baseline: 8.7936x; 1.0192x over previous
import jax
import jax.numpy as jnp
from jax import lax
from jax.experimental import pallas as pl
from jax.experimental.pallas import tpu as pltpu
from jax.experimental.pallas import tpu_sc as plsc

STU_NUM = 100000
PP_DIM = 128
BATCH = 16384
X_DIM = 128
OUT_DIM = PP_DIM + X_DIM

NC = 2
NS = 16
NW = NC * NS
B_PER_W = BATCH // NW

C = 128
NCHUNK = B_PER_W // C
NBUF = 3

XCHUNK = 4
XROWS = BATCH // NC // XCHUNK


def _gather_concat(x_hbm, idx_hbm, table_hbm, out_hbm,
                   idx_v, rows_v, spx, isem, gsem, rsem, xisem, xosem):
    cid = lax.axis_index("c")
    sid = lax.axis_index("s")
    wid = sid * NC + cid
    base = wid * B_PER_W

    def x_in(k):
        row0 = (cid * XCHUNK + k) * XROWS
        return pltpu.make_async_copy(
            x_hbm.at[pl.ds(row0, XROWS)], spx.at[k], xisem)

    def x_out(k):
        row0 = (cid * XCHUNK + k) * XROWS
        return pltpu.make_async_copy(
            spx.at[k],
            out_hbm.at[pl.ds(row0, XROWS), pl.ds(PP_DIM, X_DIM)], xosem)

    @pl.when(sid == 0)
    def _():
        for k in range(XCHUNK):
            x_in(k).start()

    for k in range(NCHUNK):
        pltpu.async_copy(idx_hbm.at[pl.ds(base + k * C, C)], idx_v.at[k],
                         isem)
    for k in range(NCHUNK):
        pltpu.make_async_copy(idx_hbm.at[pl.ds(base + k * C, C)],
                              idx_v.at[k], isem).wait()

    def gather(k, b):
        return pltpu.make_async_copy(table_hbm.at[idx_v.at[k]],
                                     rows_v.at[b], gsem.at[b])

    def rows_write(k, b):
        return pltpu.make_async_copy(
            rows_v.at[b],
            out_hbm.at[pl.ds(base + k * C, C), pl.ds(0, PP_DIM)],
            rsem.at[b])

    for k in range(min(NBUF, NCHUNK)):
        gather(k, k % NBUF).start()

    @pl.when(sid == 0)
    def _():
        for k in range(XCHUNK):
            x_in(k).wait()
            x_out(k).start()

    for k in range(NCHUNK):
        b = k % NBUF
        gather(k, b).wait()
        rows_write(k, b).start()
        nk = k + NBUF
        if nk < NCHUNK:
            rows_write(k, b).wait()
            gather(nk, b).start()
    for k in range(max(0, NCHUNK - NBUF), NCHUNK):
        rows_write(k, k % NBUF).wait()

    @pl.when(sid == 0)
    def _():
        for k in range(XCHUNK):
            x_out(k).wait()


@jax.jit
def _run(x, stu_id, ability_emb):
    mesh = plsc.VectorSubcoreMesh(core_axis_name="c", subcore_axis_name="s")
    return pl.kernel(
        _gather_concat,
        out_type=jax.ShapeDtypeStruct((BATCH, OUT_DIM), jnp.float32),
        mesh=mesh,
        scratch_types=[
            pltpu.VMEM((NCHUNK, C), jnp.int32),
            pltpu.VMEM((NBUF, C, PP_DIM), jnp.float32),
            pltpu.VMEM_SHARED((XCHUNK, XROWS, X_DIM), jnp.float32),
            pltpu.SemaphoreType.DMA,
            pltpu.SemaphoreType.DMA((NBUF,)),
            pltpu.SemaphoreType.DMA((NBUF,)),
            pltpu.SemaphoreType.DMA,
            pltpu.SemaphoreType.DMA,
        ],
    )(x, stu_id, ability_emb)


def kernel(x, stu_id, ability_emb):
    return _run(x, stu_id.astype(jnp.int32), ability_emb)
